# Initial kernel scaffold; baseline (speedup 1.0000x reference)
import jax
import jax.numpy as jnp
from jax.experimental import pallas as pl


def kernel(x, x2, batch, batch2, params):
    raise NotImplementedError("write your pallas kernel here")


# speedup vs baseline: 10.1610x; 10.1610x over previous
import functools
import numpy as np
import jax
import jax.numpy as jnp
from jax import lax
from jax.experimental import pallas as pl

_EPS = 1e-5
_SQ = np.sqrt(np.float32(1.0) + np.float32(_EPS)).astype(np.float32)
_NCLOUD = 16


def _glane(src, idx, ntiles):
    lo = jnp.bitwise_and(idx, 127)
    hi = jnp.right_shift(idx, 7)
    acc = jnp.zeros(idx.shape, src.dtype)
    for h in range(ntiles):
        g = jnp.take_along_axis(src[:, h * 128:(h + 1) * 128], lo, axis=1)
        acc = jnp.where(hi == h, g, acc)
    return acc


def _cumlanes(x):
    r, p = x.shape
    s = 1
    while s < p:
        x = x + jnp.concatenate(
            [jnp.zeros((r, s), x.dtype), x[:, :p - s]], axis=1)
        s *= 2
    return x


def _fps_body(px_ref, py_ref, pz_ref,
              idx1_ref, p1x_ref, p1y_ref, p1z_ref,
              idx2_ref, p2x_ref, p2y_ref, p2z_ref):
    px, py, pz = px_ref[...], py_ref[...], pz_ref[...]
    nc = px.shape[0]

    def fps(qx, qy, qz, s_out, ntiles):
        p = qx.shape[1]
        lane = lax.broadcasted_iota(jnp.int32, (nc, p), 1)
        lane_s = lax.broadcasted_iota(jnp.int32, (nc, s_out), 1)

        def step(s, carry):
            last, mind, sel = carry
            lx = _glane(qx, last, ntiles)
            ly = _glane(qy, last, ntiles)
            lz = _glane(qz, last, ntiles)
            dx, dy, dz = qx - lx, qy - ly, qz - lz
            d = dx * dx + dy * dy
            d = d + dz * dz
            mind = jnp.minimum(mind, d)
            m = jnp.max(mind, axis=1, keepdims=True)
            nxt = jnp.min(jnp.where(mind == m, lane, p), axis=1,
                          keepdims=True).astype(jnp.int32)
            sel = jnp.where(lane_s == s, nxt, sel)
            return nxt, mind, sel

        init = (jnp.zeros((nc, 1), jnp.int32),
                jnp.full((nc, p), jnp.inf, jnp.float32),
                jnp.zeros((nc, s_out), jnp.int32))
        _, _, sel = lax.fori_loop(1, s_out, step, init)
        return sel

    idx1 = fps(px, py, pz, 512, 8)
    p1x = _glane(px, idx1, 8)
    p1y = _glane(py, idx1, 8)
    p1z = _glane(pz, idx1, 8)
    idx2 = fps(p1x, p1y, p1z, 128, 4)
    idx1_ref[...] = idx1
    p1x_ref[...], p1y_ref[...], p1z_ref[...] = p1x, p1y, p1z
    idx2_ref[...] = idx2
    p2x_ref[...] = _glane(p1x, idx2, 4)
    p2y_ref[...] = _glane(p1y, idx2, 4)
    p2z_ref[...] = _glane(p1z, idx2, 4)


def _run_fps(px, py, pz):
    nc = px.shape[0]
    outs = [
        jax.ShapeDtypeStruct((nc, 512), jnp.int32),
        jax.ShapeDtypeStruct((nc, 512), jnp.float32),
        jax.ShapeDtypeStruct((nc, 512), jnp.float32),
        jax.ShapeDtypeStruct((nc, 512), jnp.float32),
        jax.ShapeDtypeStruct((nc, 128), jnp.int32),
        jax.ShapeDtypeStruct((nc, 128), jnp.float32),
        jax.ShapeDtypeStruct((nc, 128), jnp.float32),
        jax.ShapeDtypeStruct((nc, 128), jnp.float32),
    ]
    return pl.pallas_call(_fps_body, out_shape=outs)(px, py, pz)


def _select_body(px_ref, py_ref, pz_ref, cx_ref, cy_ref, cz_ref,
                 nidx_ref, am_ref, *, p, c, r2, k):
    px, py, pz = px_ref[0], py_ref[0], pz_ref[0]
    cx, cy, cz = cx_ref[...], cy_ref[...], cz_ref[...]
    dx, dy, dz = cx - px, cy - py, cz - pz
    d2 = dx * dx + dy * dy
    d2 = d2 + dz * dz
    bits = lax.bitcast_convert_type(d2, jnp.int32)

    def bstep(_, lh):
        lo, hi = lh
        mid = lax.shift_right_arithmetic(lo + hi, 1)
        cnt = jnp.sum((bits <= mid).astype(jnp.int32), axis=1, keepdims=True)
        ge = cnt >= k
        return jnp.where(ge, lo, mid), jnp.where(ge, mid, hi)

    lo0 = jnp.full((c, 1), -1, jnp.int32)
    hi0 = jnp.full((c, 1), 0x40800000, jnp.int32)
    _, t = lax.fori_loop(0, 31, bstep, (lo0, hi0))
    n_lt = jnp.sum((bits < t).astype(jnp.int32), axis=1, keepdims=True)
    eq = bits == t
    cum_eq = _cumlanes(eq.astype(jnp.int32))
    sel = (bits < t) | (eq & (cum_eq <= (k - n_lt)))
    fin = sel & (d2 <= r2)
    cs = _cumlanes(fin.astype(jnp.int32))
    nv = cs[:, p - 1:p]
    lane_k = lax.broadcasted_iota(jnp.int32, (c, k), 1)

    def estep(kk, nid):
        cntk = jnp.sum((cs <= kk).astype(jnp.int32), axis=1, keepdims=True)
        return jnp.where(lane_k == kk, jnp.minimum(cntk, p - 1), nid)

    nid = lax.fori_loop(0, k, estep, jnp.zeros((c, k), jnp.int32))
    am = jnp.where(lane_k < nv, 0.0, -jnp.inf).astype(jnp.float32)
    nidx_ref[...] = nid.reshape(1, c, k)
    am_ref[...] = am.reshape(1, c, k)


def _run_select(px, py, pz, cx, cy, cz, p, c, r2, k=64):
    nc = px.shape[0]
    body = functools.partial(_select_body, p=p, c=c, r2=np.float32(r2), k=k)
    pt_spec = pl.BlockSpec((1, 1, p), lambda i: (i, 0, 0))
    ct_spec = pl.BlockSpec((c, 1), lambda i: (i, 0))
    return pl.pallas_call(
        body,
        grid=(nc,),
        in_specs=[pt_spec] * 3 + [ct_spec] * 3,
        out_specs=[pl.BlockSpec((1, c, k), lambda i: (i, 0, 0))] * 2,
        out_shape=[
            jax.ShapeDtypeStruct((nc, c, k), jnp.int32),
            jax.ShapeDtypeStruct((nc, c, k), jnp.float32),
        ],
    )(px.reshape(nc, 1, p), py.reshape(nc, 1, p), pz.reshape(nc, 1, p),
      cx.reshape(nc * c, 1), cy.reshape(nc * c, 1), cz.reshape(nc * c, 1))


def _fold_body(in_ref, w_ref, out_ref):
    out_ref[...] = lax.dot_general(
        w_ref[...], in_ref[...], (((1,), (0,)), ((), ())),
        preferred_element_type=jnp.float32)


def _run_fold(in_cm, w):
    cout = w.shape[0]
    n = in_cm.shape[1]
    return pl.pallas_call(
        _fold_body,
        out_shape=jax.ShapeDtypeStruct((cout, n), jnp.float32),
    )(in_cm, w)


def _epilogue(h, b, g, be):
    h = jnp.maximum(h + b, 0.0)
    return g * (h / _SQ) + be


def _sa_mlp_body(zt_ref, ct_ref, nidx_ref, am_ref, *refs, p, cb, k, ch1,
                 nlayers):
    wrefs, out_ref = refs[:-1], refs[-1]
    zt = zt_ref[0]
    ct = ct_ref[0]
    idxf = nidx_ref[...].reshape(1, cb * k)
    idx = jnp.broadcast_to(idxf, (ch1, cb * k))
    acc = _glane(zt, idx, p // 128)
    crep = jnp.broadcast_to(ct[:, :, None], (ch1, cb, k)).reshape(ch1, cb * k)
    h = acc - crep
    b1, g1, be1 = wrefs[0][...], wrefs[1][...], wrefs[2][...]
    h = _epilogue(h, b1, g1, be1)
    wi = 3
    for _ in range(nlayers):
        w = wrefs[wi][...]
        b, g, be = wrefs[wi + 1][...], wrefs[wi + 2][...], wrefs[wi + 3][...]
        wi += 4
        h = lax.dot_general(w, h, (((1,), (0,)), ((), ())),
                            preferred_element_type=jnp.float32)
        h = _epilogue(h, b, g, be)
    h = h + am_ref[...].reshape(1, cb * k)
    chout = h.shape[0]
    out = h.reshape(chout, cb, k).max(axis=2)
    out_ref[...] = out.reshape(1, chout, cb)


def _run_sa_mlp(zt, ct, nidx, am, layer1_bgb, layers, p, c, cb, k=64):
    nc, ch1 = zt.shape[0], zt.shape[1]
    chout = layers[-1][0].shape[0]
    nblk = c // cb
    wargs = []
    for a in layer1_bgb:
        wargs.append(a.reshape(-1, 1))
    for (w, b, g, be) in layers:
        wargs += [w, b.reshape(-1, 1), g.reshape(-1, 1), be.reshape(-1, 1)]
    wspecs = [pl.BlockSpec(a.shape, lambda i, j: (0, 0)) for a in wargs]
    body = functools.partial(_sa_mlp_body, p=p, cb=cb, k=k, ch1=ch1,
                             nlayers=len(layers))
    return pl.pallas_call(
        body,
        grid=(nc, nblk),
        in_specs=[
            pl.BlockSpec((1, ch1, p), lambda i, j: (i, 0, 0)),
            pl.BlockSpec((1, ch1, cb), lambda i, j: (i, 0, j)),
            pl.BlockSpec((1, cb, k), lambda i, j: (i, j, 0)),
            pl.BlockSpec((1, cb, k), lambda i, j: (i, j, 0)),
        ] + wspecs,
        out_specs=pl.BlockSpec((1, chout, cb), lambda i, j: (i, 0, j)),
        out_shape=jax.ShapeDtypeStruct((nc, chout, c), jnp.float32),
    )(zt, ct, nidx, am, *wargs)


def _gmlp_body(xt_ref, *refs, nlayers):
    wrefs, out_ref = refs[:-1], refs[-1]
    h = xt_ref[0]
    wi = 0
    for _ in range(nlayers):
        w = wrefs[wi][...]
        b, g, be = wrefs[wi + 1][...], wrefs[wi + 2][...], wrefs[wi + 3][...]
        wi += 4
        h = lax.dot_general(w, h, (((1,), (0,)), ((), ())),
                            preferred_element_type=jnp.float32)
        h = _epilogue(h, b, g, be)
    m = jnp.max(h, axis=1, keepdims=True)
    out_ref[...] = jnp.transpose(m, (1, 0)).reshape(1, 1, -1)


def _run_gmlp(xt, layers):
    nc, chin, n = xt.shape
    chout = layers[-1][0].shape[0]
    wargs = []
    for (w, b, g, be) in layers:
        wargs += [w, b.reshape(-1, 1), g.reshape(-1, 1), be.reshape(-1, 1)]
    wspecs = [pl.BlockSpec(a.shape, lambda i: (0, 0)) for a in wargs]
    body = functools.partial(_gmlp_body, nlayers=len(layers))
    res = pl.pallas_call(
        body,
        grid=(nc,),
        in_specs=[pl.BlockSpec((1, chin, n), lambda i: (i, 0, 0))] + wspecs,
        out_specs=pl.BlockSpec((1, 1, chout), lambda i: (i, 0, 0)),
        out_shape=jax.ShapeDtypeStruct((nc, 1, chout), jnp.float32),
    )(xt, *wargs)
    return res.reshape(nc, chout)


def _head_body(g_ref, w1_ref, b1_ref, w2_ref, b2_ref, w3_ref, b3_ref,
               out_ref):
    g = g_ref[...]
    h = g[0:8] + g[8:16]
    h = jnp.maximum(
        lax.dot_general(h, w1_ref[...], (((1,), (1,)), ((), ())),
                        preferred_element_type=jnp.float32) + b1_ref[...], 0.0)
    h = jnp.maximum(
        lax.dot_general(h, w2_ref[...], (((1,), (1,)), ((), ())),
                        preferred_element_type=jnp.float32) + b2_ref[...], 0.0)
    z = lax.dot_general(h, w3_ref[...], (((1,), (1,)), ((), ())),
                        preferred_element_type=jnp.float32) + b3_ref[...]
    m = jnp.max(z, axis=1, keepdims=True)
    sh = z - m
    out_ref[...] = sh - jnp.log(jnp.sum(jnp.exp(sh), axis=1, keepdims=True))


def _run_head(gcat, w1, b1, w2, b2, w3, b3):
    return pl.pallas_call(
        _head_body,
        out_shape=jax.ShapeDtypeStruct((8, 5), jnp.float32),
    )(gcat, w1, b1.reshape(1, -1), w2, b2.reshape(1, -1), w3,
      b3.reshape(1, -1))


def kernel(x, x2, batch, batch2, params):
    nc = _NCLOUD
    xa = jnp.concatenate([x.reshape(8, 1024, 6), x2.reshape(8, 1024, 6)], 0)
    pos = xa[:, :, 0:3]
    feat = xa[:, :, 3:6]
    px, py, pz = pos[:, :, 0], pos[:, :, 1], pos[:, :, 2]

    (idx1, p1x, p1y, p1z, idx2, p2x, p2y, p2z) = _run_fps(px, py, pz)

    sa1 = params["sa1"]
    sa2 = params["sa2"]

    nidx1, am1 = _run_select(px, py, pz, p1x, p1y, p1z,
                             p=1024, c=512, r2=0.2 * 0.2)
    featT = feat.reshape(nc * 1024, 3).T
    posT = pos.reshape(nc * 1024, 3).T
    in_cm = jnp.concatenate([featT, posT], 0)
    w1 = sa1[0][0]
    z1 = _run_fold(in_cm, w1)
    p1t = jnp.stack([p1x, p1y, p1z], 0).reshape(3, nc * 512)
    c1 = _run_fold(p1t, w1[:, 3:6])
    z1t = z1.reshape(64, nc, 1024).transpose(1, 0, 2)
    c1t = c1.reshape(64, nc, 512).transpose(1, 0, 2)
    x1t = _run_sa_mlp(z1t, c1t, nidx1, am1,
                      layer1_bgb=(sa1[0][1], sa1[0][2], sa1[0][3]),
                      layers=[sa1[1], sa1[2]],
                      p=1024, c=512, cb=128)

    nidx2, am2 = _run_select(p1x, p1y, p1z, p2x, p2y, p2z,
                             p=512, c=128, r2=0.4 * 0.4)
    x1cm = x1t.transpose(1, 0, 2).reshape(128, nc * 512)
    w2 = sa2[0][0]
    z2 = _run_fold(jnp.concatenate([x1cm, p1t], 0), w2)
    p2t = jnp.stack([p2x, p2y, p2z], 0).reshape(3, nc * 128)
    c2 = _run_fold(p2t, w2[:, 128:131])
    z2t = z2.reshape(128, nc, 512).transpose(1, 0, 2)
    c2t = c2.reshape(128, nc, 128).transpose(1, 0, 2)
    xqt = _run_sa_mlp(z2t, c2t, nidx2, am2,
                      layer1_bgb=(sa2[0][1], sa2[0][2], sa2[0][3]),
                      layers=[sa2[1], sa2[2]],
                      p=512, c=128, cb=128)

    p1t3 = jnp.stack([p1x, p1y, p1z], 1)
    p2t3 = jnp.stack([p2x, p2y, p2z], 1)
    g1 = _run_gmlp(jnp.concatenate([x1t, p1t3], 1), params["sa1g"])
    g3 = _run_gmlp(jnp.concatenate([xqt, p2t3], 1), params["sa3"])

    gcat = jnp.concatenate([g1, g3], 1)
    (w1h, b1h) = params["lin1"]
    (w2h, b2h) = params["lin2"]
    (w3h, b3h) = params["lin3"]
    return _run_head(gcat, w1h, b1h, w2h, b2h, w3h, b3h)
